# Initial kernel scaffold; baseline (speedup 1.0000x reference)
import jax
import jax.numpy as jnp
from jax.experimental import pallas as pl


def kernel(inputs):
    raise NotImplementedError("write your pallas kernel here")


# speedup vs baseline: 6.3845x; 6.3845x over previous
import functools

import jax
import jax.numpy as jnp
from jax import lax
from jax.experimental import pallas as pl
from jax.experimental.pallas import tpu as pltpu
from jax.experimental.pallas import tpu_sc as plsc

B = 128
N = 32768
NV = N // 16
K = 50
NBINS = 4096
CAP = 512
ROWS_PER_W = B // 32
INT_MIN = -(2 ** 31)


def _keys(v):
  b = plsc.bitcast(v, jnp.int32)
  return jnp.where(b < 0, b ^ 0x7FFFFFFF, b)


def _extract(vec, lane):
  io = lax.iota(jnp.int32, 16)
  return jnp.sum(jnp.where(io == lane, vec, jnp.zeros_like(vec)))


def _sc_body(in_hbm, out_hbm, data, hist, ck, ci, rk, sk, si, orow):
  io = lax.iota(jnp.int32, 16)
  ones = jnp.ones((16,), jnp.int32)
  zeros = jnp.zeros((16,), jnp.int32)
  wid = lax.axis_index("c") * 16 + lax.axis_index("s")

  def row_body(r, _):
    row = wid * ROWS_PER_W + r
    pltpu.sync_copy(in_hbm.at[row], data)

    def z_body(i, _):
      hist[pl.ds(i * 16, 16)] = zeros
      return 0
    lax.fori_loop(0, (NBINS + 256) // 16, z_body, 0)

    def h_body(i, _):
      key = _keys(data[pl.ds(i * 16, 16)])
      bn = (key >> 20) + 2048
      addr = ((bn & 15) << 8) | (bn >> 4)
      plsc.addupdate_scatter(hist, [addr], ones)
      return 0
    lax.fori_loop(0, NV, h_body, 0)

    def c_body(t, _):
      acc = zeros
      for m in range(16):
        acc = acc + hist[pl.ds(m * 256 + t * 16, 16)]
      hist[pl.ds(NBINS + t * 16, 16)] = acc
      return 0
    lax.fori_loop(0, 16, c_body, 0)

    def s_body(j, carry):
      s_above, found, g_c, s_at = carry
      t = 15 - j
      cv = hist[pl.ds(NBINS + t * 16, 16)]
      tt = jnp.sum(cv)
      c = plsc.cumsum(cv)
      suf = s_above + tt - c + cv
      mask = suf >= K
      npop = jnp.sum(mask.astype(jnp.int32))
      cross = (npop > 0) & (found == 0)
      l = npop - 1
      cl = _extract(c, l)
      g_new = t * 16 + l
      s_at_new = s_above + tt - cl
      g_c = jnp.where(cross, g_new, g_c)
      s_at = jnp.where(cross, s_at_new, s_at)
      found2 = jnp.where(cross, 1, found)
      s_above = jnp.where(found == 0, s_above + tt, s_above)
      return s_above, found2, g_c, s_at
    _, _, g_c, s_at = lax.fori_loop(
        0, 16, s_body, (jnp.int32(0), jnp.int32(0), jnp.int32(0), jnp.int32(0)))

    fv = plsc.load_gather(hist, [io * 256 + g_c])
    c2 = plsc.cumsum(fv)
    t2 = jnp.sum(fv)
    suf2 = s_at + t2 - c2 + fv
    npop2 = jnp.sum((suf2 >= K).astype(jnp.int32))
    d1 = g_c * 16 + (npop2 - 1)
    thr = (d1 - 2048) * (1 << 20)

    def a_body(i, acc):
      key = _keys(data[pl.ds(i * 16, 16)])
      return acc + (key >= thr).astype(jnp.int32)
    acc = lax.fori_loop(0, NV, a_body, zeros)
    base = plsc.cumsum(acc) - acc
    n_ge = jnp.sum(acc)

    pad_k = jnp.full((16,), INT_MIN, jnp.int32)
    pad_i = jnp.full((16,), 2 ** 30, jnp.int32)
    def p_body(i, _):
      ck[pl.ds(i * 16, 16)] = pad_k
      ci[pl.ds(i * 16, 16)] = pad_i
      rk[pl.ds(i * 16, 16)] = zeros
      return 0
    lax.fori_loop(0, CAP // 16, p_body, 0)

    def b_body(i, offs):
      key = _keys(data[pl.ds(i * 16, 16)])
      ge = key >= thr
      pos = base + offs
      m = ge & (pos < CAP)
      idxv = i * 16 + io
      plsc.store_scatter(ck, [pos], key, mask=m)
      plsc.store_scatter(ci, [pos], idxv, mask=m)
      return offs + ge.astype(jnp.int32)
    lax.fori_loop(0, NV, b_body, zeros)

    n_pos = jnp.minimum(n_ge, CAP)
    nev = (n_pos + 15) >> 4

    def j_body(j, _):
      jv = (j >> 4) << 4
      lane = j & 15
      kv = ck[pl.ds(jv, 16)]
      iv = ci[pl.ds(jv, 16)]
      kj = _extract(kv, lane)
      ij = _extract(iv, lane)
      def e_body(e, _):
        ke = ck[pl.ds(e * 16, 16)]
        ie = ci[pl.ds(e * 16, 16)]
        r = rk[pl.ds(e * 16, 16)]
        add = (kj > ke).astype(jnp.int32) + ((kj == ke) & (ij < ie)).astype(jnp.int32)
        rk[pl.ds(e * 16, 16)] = r + add
        return 0
      lax.fori_loop(0, nev, e_body, 0)
      return 0
    lax.fori_loop(0, n_pos, j_body, 0)

    def w_body(e, _):
      r = rk[pl.ds(e * 16, 16)]
      m = r < K
      plsc.store_scatter(sk, [r], ck[pl.ds(e * 16, 16)], mask=m)
      plsc.store_scatter(si, [r], ci[pl.ds(e * 16, 16)], mask=m)
      return 0
    lax.fori_loop(0, nev, w_body, 0)

    k0 = sk[pl.ds(0, 16)]
    i0f = si[pl.ds(0, 16)].astype(jnp.float32)
    v0 = plsc.bitcast(jnp.where(k0 < 0, k0 ^ 0x7FFFFFFF, k0), jnp.float32)
    m10 = io < 10
    zf = jnp.zeros((16,), jnp.float32)
    mean10 = jnp.sum(jnp.where(m10, i0f, zf)) * jnp.float32(0.1)
    ms10 = jnp.sum(jnp.where(m10, v0 * v0, zf)) * jnp.float32(0.1)
    yb = 0x5F3759DF - (plsc.bitcast(jnp.full((16,), ms10), jnp.int32) >> 1)
    y = plsc.bitcast(yb, jnp.float32)
    x2 = 0.5 * ms10
    for _ in range(3):
      y = y * (1.5 - x2 * y * y)
    rms10 = jnp.where(ms10 > 0, ms10 * jnp.sum(jnp.where(io == 0, y, zf)), 0.0)
    maxf = jnp.sum(jnp.where(io == 0, i0f, zf))
    maxr = jnp.sum(jnp.where(io == 0, jnp.abs(v0), zf))
    stats = jnp.where(io == 0, mean10,
            jnp.where(io == 1, rms10,
            jnp.where(io == 2, maxf,
            jnp.where(io == 3, maxr, zf))))
    plsc.store_scatter(orow, [io], stats, mask=io < 4)
    for t in range(4):
      idxf = si[pl.ds(t * 16, 16)].astype(jnp.float32)
      plsc.store_scatter(orow, [io + (4 + 16 * t)], idxf)
    pltpu.sync_copy(orow.at[pl.ds(0, 64)], out_hbm.at[row])
    return 0

  lax.fori_loop(0, ROWS_PER_W, row_body, 0)


@jax.jit
def kernel(inputs):
  mesh = plsc.VectorSubcoreMesh(
      core_axis_name="c", subcore_axis_name="s", num_cores=2, num_subcores=16)
  f = pl.kernel(
      _sc_body,
      out_type=jax.ShapeDtypeStruct((B, 64), jnp.float32),
      mesh=mesh,
      compiler_params=pltpu.CompilerParams(
          needs_layout_passes=False, use_tc_tiling_on_sc=False),
      scratch_types=[
          pltpu.VMEM((N,), jnp.float32),
          pltpu.VMEM((NBINS + 256,), jnp.int32),
          pltpu.VMEM((CAP,), jnp.int32),
          pltpu.VMEM((CAP,), jnp.int32),
          pltpu.VMEM((CAP,), jnp.int32),
          pltpu.VMEM((64,), jnp.int32),
          pltpu.VMEM((64,), jnp.int32),
          pltpu.VMEM((80,), jnp.float32),
      ],
  )
  out = f(inputs)
  return out[:, :54]


# speedup vs baseline: 6.9471x; 1.0881x over previous
import functools

import jax
import jax.numpy as jnp
from jax import lax
from jax.experimental import pallas as pl
from jax.experimental.pallas import tpu as pltpu
from jax.experimental.pallas import tpu_sc as plsc

B = 128
N = 32768
NV = N // 16
K = 50
NBINS = 4096
CAP = 512
ROWS_PER_W = B // 32
INT_MIN = -(2 ** 31)


def _keys(v):
  b = plsc.bitcast(v, jnp.int32)
  return jnp.where(b < 0, b ^ 0x7FFFFFFF, b)


def _extract(vec, lane):
  io = lax.iota(jnp.int32, 16)
  return jnp.sum(jnp.where(io == lane, vec, jnp.zeros_like(vec)))


def _sc_body(in_hbm, out_hbm, data, hist, ck, ci, rk, sk, si, orow):
  io = lax.iota(jnp.int32, 16)
  ones = jnp.ones((16,), jnp.int32)
  zeros = jnp.zeros((16,), jnp.int32)
  wid = lax.axis_index("c") * 16 + lax.axis_index("s")

  def row_body(r, _):
    row = wid * ROWS_PER_W + r
    pltpu.sync_copy(in_hbm.at[row], data)

    def z_body(i, _):
      for u in range(4):
        hist[pl.ds((i * 4 + u) * 16, 16)] = zeros
      return 0
    lax.fori_loop(0, (NBINS + 256) // 64, z_body, 0)

    def h_body(i, _):
      for u in range(8):
        key = _keys(data[pl.ds((i * 8 + u) * 16, 16)])
        bn = (key >> 20) + 2048
        addr = ((bn & 15) << 8) | (bn >> 4)
        plsc.addupdate_scatter(hist, [addr], ones)
      return 0
    lax.fori_loop(0, NV // 8, h_body, 0)

    def c_body(t, _):
      acc = zeros
      for m in range(16):
        acc = acc + hist[pl.ds(m * 256 + t * 16, 16)]
      hist[pl.ds(NBINS + t * 16, 16)] = acc
      return 0
    lax.fori_loop(0, 16, c_body, 0)

    def s_body(j, carry):
      s_above, found, g_c, s_at = carry
      t = 15 - j
      cv = hist[pl.ds(NBINS + t * 16, 16)]
      tt = jnp.sum(cv)
      c = plsc.cumsum(cv)
      suf = s_above + tt - c + cv
      mask = suf >= K
      npop = jnp.sum(mask.astype(jnp.int32))
      cross = (npop > 0) & (found == 0)
      l = npop - 1
      cl = _extract(c, l)
      g_new = t * 16 + l
      s_at_new = s_above + tt - cl
      g_c = jnp.where(cross, g_new, g_c)
      s_at = jnp.where(cross, s_at_new, s_at)
      found2 = jnp.where(cross, 1, found)
      s_above = jnp.where(found == 0, s_above + tt, s_above)
      return s_above, found2, g_c, s_at
    _, _, g_c, s_at = lax.fori_loop(
        0, 16, s_body, (jnp.int32(0), jnp.int32(0), jnp.int32(0), jnp.int32(0)))

    fv = plsc.load_gather(hist, [io * 256 + g_c])
    c2 = plsc.cumsum(fv)
    t2 = jnp.sum(fv)
    suf2 = s_at + t2 - c2 + fv
    npop2 = jnp.sum((suf2 >= K).astype(jnp.int32))
    d1 = g_c * 16 + (npop2 - 1)
    thr = (d1 - 2048) * (1 << 20)

    def a_body(i, acc):
      for u in range(8):
        key = _keys(data[pl.ds((i * 8 + u) * 16, 16)])
        acc = acc + (key >= thr).astype(jnp.int32)
      return acc
    acc = lax.fori_loop(0, NV // 8, a_body, zeros)
    base = plsc.cumsum(acc) - acc
    n_ge = jnp.sum(acc)

    pad_k = jnp.full((16,), INT_MIN, jnp.int32)
    pad_i = jnp.full((16,), 2 ** 30, jnp.int32)
    def p_body(i, _):
      for u in range(4):
        ck[pl.ds((i * 4 + u) * 16, 16)] = pad_k
        ci[pl.ds((i * 4 + u) * 16, 16)] = pad_i
        rk[pl.ds((i * 4 + u) * 16, 16)] = zeros
      return 0
    lax.fori_loop(0, CAP // 64, p_body, 0)

    def b_body(i, offs):
      for u in range(8):
        key = _keys(data[pl.ds((i * 8 + u) * 16, 16)])
        ge = key >= thr
        pos = base + offs
        m = ge & (pos < CAP)
        idxv = (i * 8 + u) * 16 + io
        plsc.store_scatter(ck, [pos], key, mask=m)
        plsc.store_scatter(ci, [pos], idxv, mask=m)
        offs = offs + ge.astype(jnp.int32)
      return offs
    lax.fori_loop(0, NV // 8, b_body, zeros)

    n_pos = jnp.minimum(n_ge, CAP)
    nev = (n_pos + 15) >> 4

    neh = (nev + 1) >> 1

    def j_body(t, _):
      j1 = t * 2
      j2 = t * 2 + 1
      kv1 = ck[pl.ds((j1 >> 4) << 4, 16)]
      iv1 = ci[pl.ds((j1 >> 4) << 4, 16)]
      kj1 = _extract(kv1, j1 & 15)
      ij1 = _extract(iv1, j1 & 15)
      kv2 = ck[pl.ds((j2 >> 4) << 4, 16)]
      iv2 = ci[pl.ds((j2 >> 4) << 4, 16)]
      v2 = j2 < n_pos
      kj2 = jnp.where(v2, _extract(kv2, j2 & 15), INT_MIN)
      ij2 = jnp.where(v2, _extract(iv2, j2 & 15), 2 ** 30)

      def e_body(eh, _):
        for u in range(2):
          e = eh * 2 + u
          ke = ck[pl.ds(e * 16, 16)]
          ie = ci[pl.ds(e * 16, 16)]
          r = rk[pl.ds(e * 16, 16)]
          add1 = (kj1 > ke).astype(jnp.int32) + (
              (kj1 == ke) & (ij1 < ie)).astype(jnp.int32)
          add2 = (kj2 > ke).astype(jnp.int32) + (
              (kj2 == ke) & (ij2 < ie)).astype(jnp.int32)
          rk[pl.ds(e * 16, 16)] = r + add1 + add2
        return 0
      lax.fori_loop(0, neh, e_body, 0)
      return 0
    lax.fori_loop(0, (n_pos + 1) >> 1, j_body, 0)

    def w_body(e, _):
      r = rk[pl.ds(e * 16, 16)]
      m = r < K
      plsc.store_scatter(sk, [r], ck[pl.ds(e * 16, 16)], mask=m)
      plsc.store_scatter(si, [r], ci[pl.ds(e * 16, 16)], mask=m)
      return 0
    lax.fori_loop(0, nev, w_body, 0)

    k0 = sk[pl.ds(0, 16)]
    i0f = si[pl.ds(0, 16)].astype(jnp.float32)
    v0 = plsc.bitcast(jnp.where(k0 < 0, k0 ^ 0x7FFFFFFF, k0), jnp.float32)
    m10 = io < 10
    zf = jnp.zeros((16,), jnp.float32)
    mean10 = jnp.sum(jnp.where(m10, i0f, zf)) * jnp.float32(0.1)
    ms10 = jnp.sum(jnp.where(m10, v0 * v0, zf)) * jnp.float32(0.1)
    yb = 0x5F3759DF - (plsc.bitcast(jnp.full((16,), ms10), jnp.int32) >> 1)
    y = plsc.bitcast(yb, jnp.float32)
    x2 = 0.5 * ms10
    for _ in range(3):
      y = y * (1.5 - x2 * y * y)
    rms10 = jnp.where(ms10 > 0, ms10 * jnp.sum(jnp.where(io == 0, y, zf)), 0.0)
    maxf = jnp.sum(jnp.where(io == 0, i0f, zf))
    maxr = jnp.sum(jnp.where(io == 0, jnp.abs(v0), zf))
    stats = jnp.where(io == 0, mean10,
            jnp.where(io == 1, rms10,
            jnp.where(io == 2, maxf,
            jnp.where(io == 3, maxr, zf))))
    plsc.store_scatter(orow, [io], stats, mask=io < 4)
    for t in range(4):
      idxf = si[pl.ds(t * 16, 16)].astype(jnp.float32)
      plsc.store_scatter(orow, [io + (4 + 16 * t)], idxf)
    pltpu.sync_copy(orow.at[pl.ds(0, 64)], out_hbm.at[row])
    return 0

  lax.fori_loop(0, ROWS_PER_W, row_body, 0)


@jax.jit
def kernel(inputs):
  mesh = plsc.VectorSubcoreMesh(
      core_axis_name="c", subcore_axis_name="s", num_cores=2, num_subcores=16)
  f = pl.kernel(
      _sc_body,
      out_type=jax.ShapeDtypeStruct((B, 64), jnp.float32),
      mesh=mesh,
      compiler_params=pltpu.CompilerParams(
          needs_layout_passes=False, use_tc_tiling_on_sc=False),
      scratch_types=[
          pltpu.VMEM((N,), jnp.float32),
          pltpu.VMEM((NBINS + 256,), jnp.int32),
          pltpu.VMEM((CAP,), jnp.int32),
          pltpu.VMEM((CAP,), jnp.int32),
          pltpu.VMEM((CAP,), jnp.int32),
          pltpu.VMEM((64,), jnp.int32),
          pltpu.VMEM((64,), jnp.int32),
          pltpu.VMEM((80,), jnp.float32),
      ],
  )
  out = f(inputs)
  return out[:, :54]


# speedup vs baseline: 13.2563x; 1.9082x over previous
import functools

import jax
import jax.numpy as jnp
from jax import lax
from jax.experimental import pallas as pl
from jax.experimental.pallas import tpu as pltpu
from jax.experimental.pallas import tpu_sc as plsc

B = 128
N = 32768
NV = N // 16
K = 50
NBINS = 4096
CAP = 512
ROWS_PER_W = B // 32
INT_MIN = -(2 ** 31)


def _keys(v):
  b = plsc.bitcast(v, jnp.int32)
  return jnp.where(b < 0, b ^ 0x7FFFFFFF, b)


def _extract(vec, lane):
  io = lax.iota(jnp.int32, 16)
  return jnp.sum(jnp.where(io == lane, vec, jnp.zeros_like(vec)))


def _sc_body(in_hbm, out_hbm, data, hist, ck, ci, rk, sk, si, orow):
  io = lax.iota(jnp.int32, 16)
  ones = jnp.ones((16,), jnp.int32)
  zeros = jnp.zeros((16,), jnp.int32)
  wid = lax.axis_index("c") * 16 + lax.axis_index("s")

  def row_body(r, _):
    row = wid * ROWS_PER_W + r
    pltpu.sync_copy(in_hbm.at[row], data)

    def z_body(i, _):
      for u in range(4):
        hist[pl.ds((i * 4 + u) * 16, 16)] = zeros
      return 0
    lax.fori_loop(0, (NBINS + 256) // 64, z_body, 0)

    def h_body(i, _):
      vs = [data[pl.ds((i * 8 + u) * 16, 16)] for u in range(8)]
      keys = [_keys(v) for v in vs]
      addrs = []
      for key in keys:
        bn = (key >> 20) + 2048
        addrs.append(((bn & 15) << 8) | (bn >> 4))
      for addr in addrs:
        plsc.addupdate_scatter(hist, [addr], ones)
      return 0
    lax.fori_loop(0, NV // 8, h_body, 0)

    def c_body(t, _):
      acc = zeros
      for m in range(16):
        acc = acc + hist[pl.ds(m * 256 + t * 16, 16)]
      hist[pl.ds(NBINS + t * 16, 16)] = acc
      return 0
    lax.fori_loop(0, 16, c_body, 0)

    def s_body(j, carry):
      s_above, found, g_c, s_at = carry
      t = 15 - j
      cv = hist[pl.ds(NBINS + t * 16, 16)]
      tt = jnp.sum(cv)
      c = plsc.cumsum(cv)
      suf = s_above + tt - c + cv
      mask = suf >= K
      npop = jnp.sum(mask.astype(jnp.int32))
      cross = (npop > 0) & (found == 0)
      l = npop - 1
      cl = _extract(c, l)
      g_new = t * 16 + l
      s_at_new = s_above + tt - cl
      g_c = jnp.where(cross, g_new, g_c)
      s_at = jnp.where(cross, s_at_new, s_at)
      found2 = jnp.where(cross, 1, found)
      s_above = jnp.where(found == 0, s_above + tt, s_above)
      return s_above, found2, g_c, s_at
    _, _, g_c, s_at = lax.fori_loop(
        0, 16, s_body, (jnp.int32(0), jnp.int32(0), jnp.int32(0), jnp.int32(0)))

    fv = plsc.load_gather(hist, [io * 256 + g_c])
    c2 = plsc.cumsum(fv)
    t2 = jnp.sum(fv)
    suf2 = s_at + t2 - c2 + fv
    npop2 = jnp.sum((suf2 >= K).astype(jnp.int32))
    d1 = g_c * 16 + (npop2 - 1)
    thr = (d1 - 2048) * (1 << 20)

    def a_body(i, acc):
      vs = [data[pl.ds((i * 8 + u) * 16, 16)] for u in range(8)]
      ges = [(_keys(v) >= thr).astype(jnp.int32) for v in vs]
      s01 = ges[0] + ges[1]
      s23 = ges[2] + ges[3]
      s45 = ges[4] + ges[5]
      s67 = ges[6] + ges[7]
      return acc + ((s01 + s23) + (s45 + s67))
    acc = lax.fori_loop(0, NV // 8, a_body, zeros)
    base = plsc.cumsum(acc) - acc
    n_ge = jnp.sum(acc)

    pad_k = jnp.full((16,), INT_MIN, jnp.int32)
    pad_i = jnp.full((16,), 2 ** 30, jnp.int32)
    def p_body(i, _):
      for u in range(4):
        ck[pl.ds((i * 4 + u) * 16, 16)] = pad_k
        ci[pl.ds((i * 4 + u) * 16, 16)] = pad_i
        rk[pl.ds((i * 4 + u) * 16, 16)] = zeros
      return 0
    lax.fori_loop(0, CAP // 64, p_body, 0)

    def b_body(i, offs):
      vs = [data[pl.ds((i * 8 + u) * 16, 16)] for u in range(8)]
      keys = [_keys(v) for v in vs]
      ges = [k >= thr for k in keys]
      for u in range(8):
        pos = base + offs
        m = ges[u] & (pos < CAP)
        idxv = (i * 8 + u) * 16 + io
        plsc.store_scatter(ck, [pos], keys[u], mask=m)
        plsc.store_scatter(ci, [pos], idxv, mask=m)
        offs = offs + ges[u].astype(jnp.int32)
      return offs
    lax.fori_loop(0, NV // 8, b_body, zeros)

    n_pos = jnp.minimum(n_ge, CAP)
    nev = (n_pos + 15) >> 4

    neh = (nev + 1) >> 1

    def j_body(t, _):
      j1 = t * 2
      j2 = t * 2 + 1
      sj1 = zeros + j1
      sj2 = zeros + j2
      kj1 = plsc.load_gather(ck, [sj1])
      ij1 = plsc.load_gather(ci, [sj1])
      v2 = j2 < n_pos
      kj2 = jnp.where(v2, plsc.load_gather(ck, [sj2]), INT_MIN)
      ij2 = jnp.where(v2, plsc.load_gather(ci, [sj2]), 2 ** 30)

      def e_body(eh, _):
        kes = [ck[pl.ds((eh * 2 + u) * 16, 16)] for u in range(2)]
        ies = [ci[pl.ds((eh * 2 + u) * 16, 16)] for u in range(2)]
        rs = [rk[pl.ds((eh * 2 + u) * 16, 16)] for u in range(2)]
        for u in range(2):
          add1 = (kj1 > kes[u]).astype(jnp.int32) + (
              (kj1 == kes[u]) & (ij1 < ies[u])).astype(jnp.int32)
          add2 = (kj2 > kes[u]).astype(jnp.int32) + (
              (kj2 == kes[u]) & (ij2 < ies[u])).astype(jnp.int32)
          rk[pl.ds((eh * 2 + u) * 16, 16)] = rs[u] + add1 + add2
        return 0
      lax.fori_loop(0, neh, e_body, 0)
      return 0
    lax.fori_loop(0, (n_pos + 1) >> 1, j_body, 0)

    def w_body(e, _):
      r = rk[pl.ds(e * 16, 16)]
      m = r < K
      plsc.store_scatter(sk, [r], ck[pl.ds(e * 16, 16)], mask=m)
      plsc.store_scatter(si, [r], ci[pl.ds(e * 16, 16)], mask=m)
      return 0
    lax.fori_loop(0, nev, w_body, 0)

    k0 = sk[pl.ds(0, 16)]
    i0f = si[pl.ds(0, 16)].astype(jnp.float32)
    v0 = plsc.bitcast(jnp.where(k0 < 0, k0 ^ 0x7FFFFFFF, k0), jnp.float32)
    m10 = io < 10
    zf = jnp.zeros((16,), jnp.float32)
    mean10 = jnp.sum(jnp.where(m10, i0f, zf)) * jnp.float32(0.1)
    ms10 = jnp.sum(jnp.where(m10, v0 * v0, zf)) * jnp.float32(0.1)
    yb = 0x5F3759DF - (plsc.bitcast(jnp.full((16,), ms10), jnp.int32) >> 1)
    y = plsc.bitcast(yb, jnp.float32)
    x2 = 0.5 * ms10
    for _ in range(3):
      y = y * (1.5 - x2 * y * y)
    rms10 = jnp.where(ms10 > 0, ms10 * jnp.sum(jnp.where(io == 0, y, zf)), 0.0)
    maxf = jnp.sum(jnp.where(io == 0, i0f, zf))
    maxr = jnp.sum(jnp.where(io == 0, jnp.abs(v0), zf))
    stats = jnp.where(io == 0, mean10,
            jnp.where(io == 1, rms10,
            jnp.where(io == 2, maxf,
            jnp.where(io == 3, maxr, zf))))
    plsc.store_scatter(orow, [io], stats, mask=io < 4)
    for t in range(4):
      idxf = si[pl.ds(t * 16, 16)].astype(jnp.float32)
      plsc.store_scatter(orow, [io + (4 + 16 * t)], idxf)
    pltpu.sync_copy(orow.at[pl.ds(0, 64)], out_hbm.at[row])
    return 0

  lax.fori_loop(0, ROWS_PER_W, row_body, 0)


@jax.jit
def kernel(inputs):
  mesh = plsc.VectorSubcoreMesh(
      core_axis_name="c", subcore_axis_name="s", num_cores=2, num_subcores=16)
  f = pl.kernel(
      _sc_body,
      out_type=jax.ShapeDtypeStruct((B, 64), jnp.float32),
      mesh=mesh,
      compiler_params=pltpu.CompilerParams(
          needs_layout_passes=False, use_tc_tiling_on_sc=False),
      scratch_types=[
          pltpu.VMEM((N,), jnp.float32),
          pltpu.VMEM((NBINS + 256,), jnp.int32),
          pltpu.VMEM((CAP,), jnp.int32),
          pltpu.VMEM((CAP,), jnp.int32),
          pltpu.VMEM((CAP,), jnp.int32),
          pltpu.VMEM((64,), jnp.int32),
          pltpu.VMEM((64,), jnp.int32),
          pltpu.VMEM((80,), jnp.float32),
      ],
  )
  out = f(inputs)
  return out[:, :54]
